# Initial kernel scaffold; baseline (speedup 1.0000x reference)
import jax
import jax.numpy as jnp
from jax.experimental import pallas as pl


def kernel(x, tables):
    raise NotImplementedError("write your pallas kernel here")


# speedup vs baseline: 32.5938x; 32.5938x over previous
import functools

import numpy as np
import jax
import jax.numpy as jnp
from jax import lax
from jax.experimental import pallas as pl
from jax.experimental.pallas import tpu as pltpu
from jax.experimental.pallas import tpu_sc as plsc

_N_LEVELS = 16
_LOG2 = 19
_HASH = 1 << _LOG2
_MASK = _HASH - 1
_B = 524288
_FDIM = 2 * _N_LEVELS
_P1 = int(np.uint32(2654435761).view(np.int32))
_P2 = 805459861
_BFAC = np.exp((np.log(512.0) - np.log(16.0)) / (_N_LEVELS - 1))
_RES = np.floor(16.0 * _BFAC ** np.arange(_N_LEVELS)).astype(np.float32)
_GRID = [float(np.float32(2.0) / np.float32(r)) for r in _RES]
_UB = [float(np.float32(2.0) / np.float32(g)) for g in _GRID]

_LANES = 16
_C = 1024

_info = plsc.get_sparse_core_info()
_NC, _NS = _info.num_cores, _info.num_subcores
_NW = _NC * _NS
_PPW = _B // _NW
_NCHUNKS = _PPW // _C

_mesh = plsc.VectorSubcoreMesh(core_axis_name="c", subcore_axis_name="s")


def _loop_i32(n, body):
    def step(i, _):
        body(i)
        return i + np.int32(1), None

    lax.scan(step, np.int32(0), None, length=n)


@functools.partial(
    pl.kernel,
    out_type=jax.ShapeDtypeStruct((_B, _FDIM), jnp.float32),
    mesh=_mesh,
    scratch_types=[
        pltpu.VMEM((3, _C), jnp.float32),
        pltpu.VMEM((8 * _C,), jnp.int32),
        pltpu.VMEM((8 * _C, 2), jnp.float32),
        pltpu.VMEM((_C,), jnp.float32),
        pltpu.VMEM((_C,), jnp.float32),
        pltpu.VMEM((_C,), jnp.float32),
        pltpu.VMEM((_C, _FDIM), jnp.float32),
        pltpu.SemaphoreType.DMA,
    ],
    compiler_params=pltpu.CompilerParams(needs_layout_passes=False, use_tc_tiling_on_sc=False),
)
def _hash_embed(xT, tab, out, x_v, idx_v, rows_v, wx_v, wy_v, wz_v, out_v, sem):
    i32 = jnp.int32
    wid = lax.axis_index("s") * i32(_NC) + lax.axis_index("c")
    base0 = wid * i32(_PPW)

    def chunk_body(ch):
        base = base0 + ch * i32(_C)
        pltpu.sync_copy(xT.at[:, pl.ds(base, _C)], x_v)

        for l in range(_N_LEVELS):
            grid = _GRID[l]
            ub = _UB[l]
            lbase = l * _HASH

            def hash_body(g, grid=grid, ub=ub, lbase=lbase):
                j0 = g * i32(_LANES)
                bl = []
                for d, wref in enumerate((wx_v, wy_v, wz_v)):
                    xd = x_v[d, pl.ds(j0, _LANES)]
                    u = (xd + 1.0) / grid
                    t = jnp.minimum(jnp.maximum(u, 0.0), ub)
                    bi = t.astype(jnp.int32)
                    wref[pl.ds(j0, _LANES)] = u - bi.astype(jnp.float32)
                    bl.append(bi)
                bx, by, bz = bl
                hx = (bx, bx + i32(1))
                hy0 = by * i32(_P1)
                hz0 = bz * i32(_P2)
                hy = (hy0, hy0 + i32(_P1))
                hz = (hz0, hz0 + i32(_P2))
                c = 0
                for i in (0, 1):
                    for j in (0, 1):
                        exy = hx[i] ^ hy[j]
                        for k in (0, 1):
                            h = ((exy ^ hz[k]) & i32(_MASK)) + i32(lbase)
                            idx_v[pl.ds(i32(c * _C) + j0, _LANES)] = h
                            c += 1

            _loop_i32(_C // _LANES, hash_body)

            pltpu.async_copy(tab.at[idx_v], rows_v, sem).wait()

            def interp_body(g, l=l):
                j0 = g * i32(_LANES)
                row = j0 + lax.iota(jnp.int32, _LANES)
                zero = jnp.zeros((_LANES,), jnp.int32)
                one = zero + i32(1)
                wx = wx_v[pl.ds(j0, _LANES)]
                wy = wy_v[pl.ds(j0, _LANES)]
                wz = wz_v[pl.ds(j0, _LANES)]
                wxt = (1.0 - wx, wx)
                wyt = (1.0 - wy, wy)
                wzt = (1.0 - wz, wz)
                acc0 = jnp.zeros((_LANES,), jnp.float32)
                acc1 = jnp.zeros((_LANES,), jnp.float32)
                c = 0
                for i in (0, 1):
                    for j in (0, 1):
                        cxy = wxt[i] * wyt[j]
                        for k in (0, 1):
                            coeff = cxy * wzt[k]
                            r = i32(c * _C) + row
                            e0 = plsc.load_gather(rows_v, [r, zero])
                            e1 = plsc.load_gather(rows_v, [r, one])
                            acc0 = acc0 + coeff * e0
                            acc1 = acc1 + coeff * e1
                            c += 1
                col0 = jnp.full((_LANES,), 2 * l, jnp.int32)
                plsc.store_scatter(out_v, [row, col0], acc0)
                plsc.store_scatter(out_v, [row, col0 + i32(1)], acc1)

            _loop_i32(_C // _LANES, interp_body)

        pltpu.sync_copy(out_v, out.at[pl.ds(base, _C), :])

    _loop_i32(_NCHUNKS, chunk_body)


def kernel(x, tables):
    xT = jnp.asarray(x, jnp.float32).T
    tab = jnp.asarray(tables, jnp.float32).reshape(_N_LEVELS * _HASH, 2)
    return _hash_embed(xT, tab)
